# Initial kernel scaffold; baseline (speedup 1.0000x reference)
import jax
import jax.numpy as jnp
from jax.experimental import pallas as pl


def kernel(x, knots, poly_params, mixture_weights, integrals_2dgrid):
    raise NotImplementedError("write your pallas kernel here")


# speedup vs baseline: 107.6053x; 107.6053x over previous
import functools

import jax
import jax.numpy as jnp
from jax import lax
from jax.experimental import pallas as pl
from jax.experimental.pallas import tpu as pltpu
from jax.experimental.pallas import tpu_sc as plsc

K = 512
B = K - 1
M = 4
DEG = 3
NCOEF = M * DEG * DEG
NROW = 48

NW = 32
CHUNK = 128
VECS = CHUNK // 16
LN2 = 0.6931471805599453


@functools.partial(jax.jit, static_argnames=("P", "CHUNKS"))
def _sc_call(x0, x1, k0, k1, tab, P, CHUNKS):
  mesh = plsc.VectorSubcoreMesh(core_axis_name="c", subcore_axis_name="s")

  @functools.partial(
      pl.kernel,
      mesh=mesh,
      compiler_params=pltpu.CompilerParams(
          needs_layout_passes=False, use_tc_tiling_on_sc=False),
      out_type=jax.ShapeDtypeStruct((NW * P,), jnp.float32),
      scratch_types=[
          pltpu.VMEM((P,), jnp.float32),
          pltpu.VMEM((P,), jnp.float32),
          pltpu.VMEM((P,), jnp.float32),
          pltpu.VMEM((K,), jnp.float32),
          pltpu.VMEM((K,), jnp.float32),
          pltpu.VMEM((CHUNK, NROW), jnp.float32),
          pltpu.VMEM((CHUNK,), jnp.int32),
          pltpu.VMEM((CHUNK,), jnp.float32),
          pltpu.VMEM((CHUNK,), jnp.float32),
          pltpu.SemaphoreType.DMA,
      ],
  )
  def kern(x0_hbm, x1_hbm, k0_hbm, k1_hbm, tab_hbm, out_hbm,
           x0_v, x1_v, out_v, k0_v, k1_v, coeff_v, idx_v, t0_v, t1_v, sem):
    wid = lax.axis_index("s") * 2 + lax.axis_index("c")
    base = wid * P
    pltpu.sync_copy(x0_hbm.at[pl.ds(base, P)], x0_v)
    pltpu.sync_copy(x1_hbm.at[pl.ds(base, P)], x1_v)
    pltpu.sync_copy(k0_hbm, k0_v)
    pltpu.sync_copy(k1_hbm, k1_v)

    lanes = lax.iota(jnp.int32, 16)

    def search(kref, xv):
      pos = jnp.zeros((16,), jnp.int32)
      step = K // 2
      while step >= 1:
        kv = plsc.load_gather(kref, [pos + (step - 1)])
        pos = jnp.where(kv < xv, pos + step, pos)
        step //= 2
      return jnp.clip(pos - 1, 0, B - 1)

    def chunk_body(g, carry):
      off = g * CHUNK

      def search_vec(jv, c):
        o = off + jv * 16
        xv0 = x0_v[pl.ds(o, 16)]
        xv1 = x1_v[pl.ds(o, 16)]
        i0 = search(k0_v, xv0)
        i1 = search(k1_v, xv1)
        s0 = plsc.load_gather(k0_v, [i0])
        s1 = plsc.load_gather(k1_v, [i1])
        w = jv * 16
        idx_v[pl.ds(w, 16)] = i0 * B + i1
        t0_v[pl.ds(w, 16)] = xv0 - s0
        t1_v[pl.ds(w, 16)] = xv1 - s1
        return c

      lax.fori_loop(0, VECS, search_vec, 0)
      pltpu.async_copy(tab_hbm.at[idx_v], coeff_v, sem).wait()

      def compute_vec(jv, c):
        w = jv * 16
        pt = lanes + w
        t0 = t0_v[pl.ds(w, 16)]
        t1 = t1_v[pl.ds(w, 16)]
        acc = jnp.zeros((16,), jnp.float32)
        for m in range(M):
          q = []
          for i in range(DEG):
            cb = m * 9 + i * 3
            c0 = plsc.load_gather(coeff_v, [pt, jnp.full((16,), cb, jnp.int32)])
            c1 = plsc.load_gather(coeff_v, [pt, jnp.full((16,), cb + 1, jnp.int32)])
            c2 = plsc.load_gather(coeff_v, [pt, jnp.full((16,), cb + 2, jnp.int32)])
            q.append(c0 + t1 * (c1 + t1 * c2))
          val = q[0] + t0 * (q[1] + t0 * q[2])
          acc = acc + val * val
        d = jnp.maximum(acc, 1e-38)
        small = d < 1.1754944e-38
        d = jnp.where(small, d * 1.8446744e19, d)
        bits = lax.bitcast_convert_type(d, jnp.int32)
        e = (bits >> 23) - 127
        mant = lax.bitcast_convert_type(
            (bits & 0x007FFFFF) | 0x3F800000, jnp.float32)
        big = mant > 1.4142135
        mant = jnp.where(big, mant * 0.5, mant)
        e = jnp.where(big, e + 1, e)
        z = (mant - 1.0) / (mant + 1.0)
        z2 = z * z
        lnm = z * (2.0 + z2 * (2.0 / 3.0 + z2 * (0.4 + z2 * (2.0 / 7.0))))
        ef = e.astype(jnp.float32) - jnp.where(small, 64.0, 0.0)
        out_v[pl.ds(off + w, 16)] = ef * LN2 + lnm
        return c

      lax.fori_loop(0, VECS, compute_vec, 0)
      return carry

    lax.fori_loop(0, CHUNKS, chunk_body, 0)
    pltpu.sync_copy(out_v, out_hbm.at[pl.ds(base, P)])

  return kern(x0, x1, k0, k1, tab)


def kernel(x, knots, poly_params, mixture_weights, integrals_2dgrid):
  n = x.shape[0]
  chunks = -(-n // (NW * CHUNK))
  p = chunks * CHUNK
  n_pad = NW * p
  scale = jnp.sqrt(mixture_weights[0][:, None, None] / integrals_2dgrid[0])
  tab = poly_params[0] * scale[:, :, :, None, None]
  tab = tab.reshape(M, B * B, DEG * DEG).transpose(1, 0, 2).reshape(B * B, NCOEF)
  tab = jnp.pad(tab, ((0, 0), (0, NROW - NCOEF)))
  x0 = jnp.pad(x[:, 0], (0, n_pad - n))
  x1 = jnp.pad(x[:, 1], (0, n_pad - n))
  out = _sc_call(x0, x1, knots[:, 0], knots[:, 1], tab, p, chunks)
  return out[:n]
